# Initial kernel scaffold; baseline (speedup 1.0000x reference)
import jax
import jax.numpy as jnp
from jax.experimental import pallas as pl


def kernel(x):
    raise NotImplementedError("write your pallas kernel here")


# speedup vs baseline: 220.5396x; 220.5396x over previous
import jax
import jax.numpy as jnp
from jax.experimental import pallas as pl

T = 16


def _body(x_ref, o_ref):
    x = x_ref[0]
    st = (jax.nn.sigmoid(x) * (T - 1)).astype(jnp.int32)
    for t in range(T):
        o_ref[0, t] = (st == t).astype(jnp.float32)


def kernel(x):
    B, S, D = x.shape
    BS = 128
    return pl.pallas_call(
        _body,
        grid=(B, S // BS),
        in_specs=[pl.BlockSpec((1, BS, D), lambda b, s: (b, s, 0))],
        out_specs=pl.BlockSpec((1, T, BS, D), lambda b, s: (b, 0, s, 0)),
        out_shape=jax.ShapeDtypeStruct((B, T, S, D), jnp.float32),
    )(x)
